# baseline (speedup 1.0000x reference)
import functools

import jax
import jax.numpy as jnp
from jax import lax
from jax.experimental import pallas as pl
from jax.experimental.pallas import tpu as pltpu
from jax.experimental.pallas import tpu_sc as plsc

_B = 4096
_T = 200
_D = 64
_N = _B * _T
_L = 16
_NC = 2
_NS = 16
_NW = _NC * _NS
_V = 1000001
_VP = 1000064
_TCH = _V // 128
_TAIL = _V - _TCH * 128
_TAILP = 80
_ASL = (_TCH + _NW - 1) // _NW

_PW = _N // _NW
_G = 128
_C = 256
_NB = 4
_Q = _PW // _C
_QG = _C // _G
_IR = _PW // _G

_TB = 8
_BW = _B // _NW


def _wid():
    return lax.axis_index("s") * _NC + lax.axis_index("c")


def _make_table_transpose():
    mesh = plsc.VectorSubcoreMesh(core_axis_name="c", subcore_axis_name="s")

    @functools.partial(
        pl.kernel,
        mesh=mesh,
        out_type=jax.ShapeDtypeStruct((_VP // 2, 128), jnp.float32),
        scratch_types=[pltpu.VMEM((_D, 128), jnp.float32) for _ in range(4)]
        + [
            pltpu.VMEM((_TAILP, _D), jnp.float32),
            pltpu.VMEM((_TAILP // 2, 128), jnp.float32),
        ]
        + [pltpu.SemaphoreType.DMA for _ in range(4)],
        compiler_params=pltpu.CompilerParams(use_tc_tiling_on_sc=True, needs_layout_passes=False),
    )
    def body(tt_hbm, tail_hbm, pairs_hbm, ib0, ib1, ob0, ob1, tl_v, tlp_v,
             is0, is1, os0, os1):
        w = _wid()
        ib, ob = [ib0, ib1], [ob0, ob1]
        isem, osem = [is0, is1], [os0, os1]
        lanes = lax.iota(jnp.int32, _L)
        rowhalf = lax.shift_right_logical(lanes, 1)
        parity64 = (lanes & 1) * _D
        rowbase = [g * 8 + rowhalf for g in range(8)]

        def chunk_of(k):
            return k * _NW + w

        def fire_in(k, b):
            c = chunk_of(k)
            pltpu.async_copy(
                tt_hbm.at[:, pl.ds(pl.multiple_of(c * 128, 128), 128)],
                ib[b],
                isem[b],
            )

        def wait_in(k, b):
            del k
            pltpu.make_async_copy(
                tt_hbm.at[:, pl.ds(0, 128)], ib[b], isem[b]
            ).wait()

        def fire_out(k, b):
            c = chunk_of(k)
            pltpu.async_copy(
                ob[b],
                pairs_hbm.at[pl.ds(pl.multiple_of(c * _D, 8), _D)],
                osem[b],
            )

        def wait_out(b):
            pltpu.make_async_copy(
                ob[b], pairs_hbm.at[pl.ds(0, _D)], osem[b]
            ).wait()

        def transpose(b):
            def dstep(ds_i, carry):
                for dd in range(8):
                    d = ds_i * 8 + dd
                    colv = parity64 + d
                    for g in range(8):
                        v = ib[b][d, pl.ds(g * _L, _L)]
                        plsc.store_scatter(ob[b], [rowbase[g], colv], v)
                return carry

            lax.fori_loop(0, 8, dstep, 0)

        fire_in(0, 0)

        def macro(gg, carry):
            for j in range(2):
                k = gg * 2 + j
                c = chunk_of(k)

                @pl.when(chunk_of(k + 1) < _TCH)
                def _():
                    fire_in(k + 1, 1 - j)

                @pl.when(c < _TCH)
                def _():
                    wait_in(k, j)

                    @pl.when(k >= 2)
                    def _():
                        wait_out(j)

                    transpose(j)
                    fire_out(k, j)
            return carry

        lax.fori_loop(0, (_ASL + 1) // 2, macro, 0)
        for j in range(2):
            wait_out(j)

        @pl.when(w == _NW - 1)
        def _():
            pltpu.sync_copy(tail_hbm, tl_v)
            for r in range(_TAILP):
                for g in range(_D // _L):
                    tlp_v[r // 2, pl.ds((r % 2) * _D + g * _L, _L)] = (
                        tl_v[r, pl.ds(g * _L, _L)]
                    )
            pltpu.sync_copy(
                tlp_v,
                pairs_hbm.at[pl.ds(pl.multiple_of(_TCH * _D, 8), _TAILP // 2)],
            )

    return body


def _make_sc_gather():
    mesh = plsc.VectorSubcoreMesh(core_axis_name="c", subcore_axis_name="s")

    @functools.partial(
        pl.kernel,
        mesh=mesh,
        out_type=jax.ShapeDtypeStruct((_N, _D), jnp.float32),
        scratch_types=[
            pltpu.VMEM((_IR, _G), jnp.int32),
            pltpu.VMEM((_L,), jnp.int32),
        ]
        + [pltpu.VMEM((_C, _D), jnp.float32) for _ in range(_NB)]
        + [pltpu.SemaphoreType.DMA for _ in range(2 * _NB)],
        compiler_params=pltpu.CompilerParams(use_tc_tiling_on_sc=False),
    )
    def body(text_hbm, seqv_hbm, table_hbm, out_hbm, idx_v, seq_v, *bufs):
        rows = list(bufs[:_NB])
        gsem = list(bufs[_NB:2 * _NB])
        wsem = list(bufs[2 * _NB:])
        base_row = _wid() * _PW

        pltpu.sync_copy(seqv_hbm, seq_v)
        seq = seq_v[...]
        lanes = lax.iota(jnp.int32, _L)

        pltpu.sync_copy(
            text_hbm.at[pl.ds(pl.multiple_of(base_row // _G, 8), _IR)], idx_v
        )

        def adj(r, carry):
            rbase = r * _G
            for i in range(_G // _L):
                v = idx_v[r, pl.ds(i * _L, _L)]
                pos = lax.rem(rbase + i * _L + lanes, _T)
                idx_v[r, pl.ds(i * _L, _L)] = jnp.where(pos < seq, v + 1, 0)
            return carry

        lax.fori_loop(0, _IR, adj, 0)

        def fire(q, b):
            for j in range(_QG):
                pltpu.async_copy(
                    table_hbm.at[idx_v.at[q * _QG + j]],
                    rows[b].at[pl.ds(j * _G, _G)],
                    gsem[b],
                )

        def wait_gathers(b):
            for j in range(_QG):
                pltpu.make_async_copy(
                    table_hbm.at[idx_v.at[0]],
                    rows[b].at[pl.ds(j * _G, _G)],
                    gsem[b],
                ).wait()

        def writeback(q, b):
            row0 = pl.multiple_of(base_row + q * _C, _C)
            pltpu.async_copy(rows[b], out_hbm.at[pl.ds(row0, _C)], wsem[b])

        def wait_writeback(b):
            pltpu.make_async_copy(
                rows[b],
                out_hbm.at[pl.ds(pl.multiple_of(base_row, _C), _C)],
                wsem[b],
            ).wait()

        fire(0, 0)

        def macro(gg, carry):
            for j in range(_NB):
                q = gg * _NB + j
                nb = (j + 1) % _NB

                @pl.when(q + 1 < _Q)
                def _():
                    @pl.when(q + 1 >= _NB)
                    def _():
                        wait_writeback(nb)

                    fire(q + 1, nb)

                wait_gathers(j)
                writeback(q, j)
            return carry

        lax.fori_loop(0, _Q // _NB, macro, 0)
        for b in range(_NB):
            wait_writeback(b)

    return body


def _make_out_transpose():
    mesh = plsc.VectorSubcoreMesh(core_axis_name="c", subcore_axis_name="s")

    @functools.partial(
        pl.kernel,
        mesh=mesh,
        out_type=jax.ShapeDtypeStruct((_T, _D, _B), jnp.float32),
        scratch_types=[pltpu.VMEM((_BW, _TB // 2, 128), jnp.float32)]
        + [pltpu.VMEM((_D, 128), jnp.float32) for _ in range(2)]
        + [pltpu.SemaphoreType.DMA for _ in range(3)],
        compiler_params=pltpu.CompilerParams(use_tc_tiling_on_sc=True, needs_layout_passes=False),
    )
    def body(cin_hbm, out_hbm, inb, ob0, ob1, isem, os0, os1):
        w = _wid()
        ob, osem = [ob0, ob1], [os0, os1]
        lanes = lax.iota(jnp.int32, _L)

        def load_block(tb):
            def per_b(i, carry):
                p0 = (w * _BW + i) * (_T // 2) + tb * (_TB // 2)
                pltpu.async_copy(
                    cin_hbm.at[pl.ds(pl.multiple_of(p0, 4), _TB // 2)],
                    inb.at[i],
                    isem,
                )
                return carry

            lax.fori_loop(0, _BW, per_b, 0)

            def drain(i, carry):
                pltpu.make_async_copy(
                    cin_hbm.at[pl.ds(0, _TB // 2)], inb.at[i], isem
                ).wait()
                return carry

            lax.fori_loop(0, _BW, drain, 0)

        def transpose_t(t_local, b):
            tp = t_local // 2
            half = (t_local % 2) * _D

            def per_b(i, carry):
                colv = jnp.broadcast_to(i, (_L,)).astype(jnp.int32)
                for g in range(_D // _L):
                    v = inb[i, tp, pl.ds(half + g * _L, _L)]
                    plsc.store_scatter(ob[b], [g * _L + lanes, colv], v)
                return carry

            lax.fori_loop(0, _BW, per_b, 0)

        def fire_out(t, b):
            pltpu.async_copy(
                ob[b],
                out_hbm.at[t, :, pl.ds(pl.multiple_of(w * _BW, 128), _BW)],
                osem[b],
            )

        def wait_out(b):
            pltpu.make_async_copy(
                ob[b],
                out_hbm.at[0, :, pl.ds(0, _BW)],
                osem[b],
            ).wait()

        def tblock(tb, carry):
            load_block(tb)
            for t_local in range(_TB):
                b = t_local % 2
                t = tb * _TB + t_local

                @pl.when(tb * _TB + t_local >= 2)
                def _():
                    wait_out(b)

                transpose_t(t_local, b)
                fire_out(t, b)
            return carry

        lax.fori_loop(0, _T // _TB, tblock, 0)
        for b in range(2):
            wait_out(b)

    return body


_table_transpose = _make_table_transpose()
_sc_gather = _make_sc_gather()
_out_transpose = _make_out_transpose()


def kernel(text, seq_len, text_embed):
    tt = jnp.swapaxes(text_embed, 0, 1)
    tail = jnp.zeros((_TAILP, _D), jnp.float32).at[:_TAIL].set(text_embed[_TCH * 128:])
    pairs = _table_transpose(tt, tail)
    tbl = pairs.reshape(_VP, _D)
    text2d = text.reshape(_N // _G, _G)
    seqv = jnp.full((_L,), seq_len, dtype=jnp.int32)
    flat = _sc_gather(text2d, seqv, tbl)
    cin = flat.reshape(_N // 2, 2 * _D)
    tout = _out_transpose(cin)
    return jnp.transpose(tout, (2, 0, 1))


# speedup vs baseline: 1.3355x; 1.3355x over previous
import functools

import jax
import jax.numpy as jnp
from jax import lax
from jax.experimental import pallas as pl
from jax.experimental.pallas import tpu as pltpu
from jax.experimental.pallas import tpu_sc as plsc

_B = 4096
_T = 200
_D = 64
_N = _B * _T
_L = 16
_NC = 2
_NS = 16
_NW = _NC * _NS
_V = 1000001
_VP = 1000064
_TCH = _V // 128
_TAIL = _V - _TCH * 128
_TAILP = 80
_ASL = (_TCH + _NW - 1) // _NW

_PW = _N // _NW
_G = 128
_C = 256
_NB = 4
_Q = _PW // _C
_QG = _C // _G
_IR = _PW // _G

_TB = 8
_BW = _B // _NW


def _wid():
    return lax.axis_index("s") * _NC + lax.axis_index("c")


def _make_table_transpose():
    mesh = plsc.VectorSubcoreMesh(core_axis_name="c", subcore_axis_name="s")

    @functools.partial(
        pl.kernel,
        mesh=mesh,
        out_type=jax.ShapeDtypeStruct((_VP // 2, 128), jnp.float32),
        scratch_types=[pltpu.VMEM((_D, 128), jnp.float32) for _ in range(4)]
        + [
            pltpu.VMEM((_TAILP, _D), jnp.float32),
            pltpu.VMEM((_TAILP // 2, 128), jnp.float32),
        ]
        + [pltpu.SemaphoreType.DMA for _ in range(4)],
        compiler_params=pltpu.CompilerParams(use_tc_tiling_on_sc=True, needs_layout_passes=False),
    )
    def body(tt_hbm, tail_hbm, pairs_hbm, ib0, ib1, ob0, ob1, tl_v, tlp_v,
             is0, is1, os0, os1):
        w = _wid()
        ib, ob = [ib0, ib1], [ob0, ob1]
        isem, osem = [is0, is1], [os0, os1]
        lanes = lax.iota(jnp.int32, _L)
        rowhalf = lax.shift_right_logical(lanes, 1)
        parity64 = (lanes & 1) * _D
        rowbase = [g * 8 + rowhalf for g in range(8)]

        def chunk_of(k):
            return k * _NW + w

        def fire_in(k, b):
            c = chunk_of(k)
            pltpu.async_copy(
                tt_hbm.at[:, pl.ds(pl.multiple_of(c * 128, 128), 128)],
                ib[b],
                isem[b],
            )

        def wait_in(k, b):
            del k
            pltpu.make_async_copy(
                tt_hbm.at[:, pl.ds(0, 128)], ib[b], isem[b]
            ).wait()

        def fire_out(k, b):
            c = chunk_of(k)
            pltpu.async_copy(
                ob[b],
                pairs_hbm.at[pl.ds(pl.multiple_of(c * _D, 8), _D)],
                osem[b],
            )

        def wait_out(b):
            pltpu.make_async_copy(
                ob[b], pairs_hbm.at[pl.ds(0, _D)], osem[b]
            ).wait()

        def transpose(b):
            def dstep(ds_i, carry):
                for dd in range(8):
                    d = ds_i * 8 + dd
                    colv = parity64 + d
                    for g in range(8):
                        v = ib[b][d, pl.ds(g * _L, _L)]
                        plsc.store_scatter(ob[b], [rowbase[g], colv], v)
                return carry

            lax.fori_loop(0, 8, dstep, 0)

        fire_in(0, 0)

        def macro(gg, carry):
            for j in range(2):
                k = gg * 2 + j
                c = chunk_of(k)

                @pl.when(chunk_of(k + 1) < _TCH)
                def _():
                    fire_in(k + 1, 1 - j)

                @pl.when(c < _TCH)
                def _():
                    wait_in(k, j)

                    @pl.when(k >= 2)
                    def _():
                        wait_out(j)

                    transpose(j)
                    fire_out(k, j)
            return carry

        lax.fori_loop(0, (_ASL + 1) // 2, macro, 0)
        for j in range(2):
            wait_out(j)

        @pl.when(w == _NW - 1)
        def _():
            pltpu.sync_copy(tail_hbm, tl_v)
            for r in range(_TAILP):
                for g in range(_D // _L):
                    tlp_v[r // 2, pl.ds((r % 2) * _D + g * _L, _L)] = (
                        tl_v[r, pl.ds(g * _L, _L)]
                    )
            pltpu.sync_copy(
                tlp_v,
                pairs_hbm.at[pl.ds(pl.multiple_of(_TCH * _D, 8), _TAILP // 2)],
            )

    return body


def _make_sc_gather():
    mesh = plsc.VectorSubcoreMesh(core_axis_name="c", subcore_axis_name="s")

    @functools.partial(
        pl.kernel,
        mesh=mesh,
        out_type=jax.ShapeDtypeStruct((_N, _D), jnp.float32),
        scratch_types=[
            pltpu.VMEM((_IR, _G), jnp.int32),
            pltpu.VMEM((_L,), jnp.int32),
        ]
        + [pltpu.VMEM((_C, _D), jnp.float32) for _ in range(_NB)]
        + [pltpu.SemaphoreType.DMA for _ in range(2 * _NB)],
        compiler_params=pltpu.CompilerParams(use_tc_tiling_on_sc=False),
    )
    def body(text_hbm, seqv_hbm, table_hbm, out_hbm, idx_v, seq_v, *bufs):
        rows = list(bufs[:_NB])
        gsem = list(bufs[_NB:2 * _NB])
        wsem = list(bufs[2 * _NB:])
        base_row = _wid() * _PW

        pltpu.sync_copy(seqv_hbm, seq_v)
        seq = seq_v[...]
        lanes = lax.iota(jnp.int32, _L)

        pltpu.sync_copy(
            text_hbm.at[pl.ds(pl.multiple_of(base_row // _G, 8), _IR)], idx_v
        )

        def adj(r, carry):
            rbase = r * _G
            for i in range(_G // _L):
                v = idx_v[r, pl.ds(i * _L, _L)]
                pos = lax.rem(rbase + i * _L + lanes, _T)
                idx_v[r, pl.ds(i * _L, _L)] = jnp.where(pos < seq, v + 1, 0)
            return carry

        lax.fori_loop(0, _IR, adj, 0)

        def fire(q, b):
            for j in range(_QG):
                pltpu.async_copy(
                    table_hbm.at[idx_v.at[q * _QG + j]],
                    rows[b].at[pl.ds(j * _G, _G)],
                    gsem[b],
                )

        def wait_gathers(b):
            for j in range(_QG):
                pltpu.make_async_copy(
                    table_hbm.at[idx_v.at[0]],
                    rows[b].at[pl.ds(j * _G, _G)],
                    gsem[b],
                ).wait()

        def writeback(q, b):
            row0 = pl.multiple_of(base_row + q * _C, _C)
            pltpu.async_copy(rows[b], out_hbm.at[pl.ds(row0, _C)], wsem[b])

        def wait_writeback(b):
            pltpu.make_async_copy(
                rows[b],
                out_hbm.at[pl.ds(pl.multiple_of(base_row, _C), _C)],
                wsem[b],
            ).wait()

        fire(0, 0)

        def macro(gg, carry):
            for j in range(_NB):
                q = gg * _NB + j
                nb = (j + 1) % _NB

                @pl.when(q + 1 < _Q)
                def _():
                    @pl.when(q + 1 >= _NB)
                    def _():
                        wait_writeback(nb)

                    fire(q + 1, nb)

                wait_gathers(j)
                writeback(q, j)
            return carry

        lax.fori_loop(0, _Q // _NB, macro, 0)
        for b in range(_NB):
            wait_writeback(b)

    return body


def _make_out_transpose():
    mesh = plsc.VectorSubcoreMesh(core_axis_name="c", subcore_axis_name="s")

    @functools.partial(
        pl.kernel,
        mesh=mesh,
        out_type=jax.ShapeDtypeStruct((_T, _D, _B), jnp.float32),
        scratch_types=[pltpu.VMEM((_BW, _TB // 2, 128), jnp.float32)]
        + [pltpu.VMEM((_D, 129), jnp.float32) for _ in range(2)]
        + [pltpu.SemaphoreType.DMA for _ in range(3)],
        compiler_params=pltpu.CompilerParams(use_tc_tiling_on_sc=True, needs_layout_passes=False),
    )
    def body(cin_hbm, out_hbm, inb, ob0, ob1, isem, os0, os1):
        w = _wid()
        ob, osem = [ob0, ob1], [os0, os1]
        lanes = lax.iota(jnp.int32, _L)

        def load_block(tb):
            def per_b(i, carry):
                p0 = (w * _BW + i) * (_T // 2) + tb * (_TB // 2)
                pltpu.async_copy(
                    cin_hbm.at[pl.ds(pl.multiple_of(p0, 4), _TB // 2)],
                    inb.at[i],
                    isem,
                )
                return carry

            lax.fori_loop(0, _BW, per_b, 0)

            def drain(i, carry):
                pltpu.make_async_copy(
                    cin_hbm.at[pl.ds(0, _TB // 2)], inb.at[i], isem
                ).wait()
                return carry

            lax.fori_loop(0, _BW, drain, 0)

        def transpose_t(t_local, b):
            tp = t_local // 2
            half = (t_local % 2) * _D

            def per_b(i, carry):
                colv = jnp.broadcast_to(i, (_L,)).astype(jnp.int32)
                for g in range(_D // _L):
                    v = inb[i, tp, pl.ds(half + g * _L, _L)]
                    plsc.store_scatter(ob[b], [g * _L + lanes, colv], v)
                return carry

            lax.fori_loop(0, _BW, per_b, 0)

        def fire_out(t, b):
            pltpu.async_copy(
                ob[b].at[:, pl.ds(0, 128)],
                out_hbm.at[t, :, pl.ds(pl.multiple_of(w * _BW, 128), _BW)],
                osem[b],
            )

        def wait_out(b):
            pltpu.make_async_copy(
                ob[b].at[:, pl.ds(0, 128)],
                out_hbm.at[0, :, pl.ds(0, _BW)],
                osem[b],
            ).wait()

        def tblock(tb, carry):
            load_block(tb)
            for t_local in range(_TB):
                b = t_local % 2
                t = tb * _TB + t_local

                @pl.when(tb * _TB + t_local >= 2)
                def _():
                    wait_out(b)

                transpose_t(t_local, b)
                fire_out(t, b)
            return carry

        lax.fori_loop(0, _T // _TB, tblock, 0)
        for b in range(2):
            wait_out(b)

    return body


_sc_gather = _make_sc_gather()
_out_transpose = _make_out_transpose()


def kernel(text, seq_len, text_embed):
    text2d = text.reshape(_N // _G, _G)
    seqv = jnp.full((_L,), seq_len, dtype=jnp.int32)
    flat = _sc_gather(text2d, seqv, text_embed)
    cin = flat.reshape(_N // 2, 2 * _D)
    tout = _out_transpose(cin)
    return jnp.transpose(tout, (2, 0, 1))


# speedup vs baseline: 1.9729x; 1.4773x over previous
import functools

import jax
import jax.numpy as jnp
from jax import lax
from jax.experimental import pallas as pl
from jax.experimental.pallas import tpu as pltpu
from jax.experimental.pallas import tpu_sc as plsc

_B = 4096
_T = 200
_D = 64
_N = _B * _T
_L = 16
_NC = 2
_NS = 16
_NW = _NC * _NS
_PW = _N // _NW
_G = 128
_C = 256
_NB = 4
_Q = _PW // _C
_QG = _C // _G
_IR = _PW // _G


def _make_sc_gather():
    mesh = plsc.VectorSubcoreMesh(core_axis_name="c", subcore_axis_name="s")

    @functools.partial(
        pl.kernel,
        mesh=mesh,
        out_type=jax.ShapeDtypeStruct((_N, _D), jnp.float32),
        scratch_types=[
            pltpu.VMEM((_IR, _G), jnp.int32),
            pltpu.VMEM((_L,), jnp.int32),
        ]
        + [pltpu.VMEM((_C, _D), jnp.float32) for _ in range(_NB)]
        + [pltpu.SemaphoreType.DMA for _ in range(2 * _NB)],
        compiler_params=pltpu.CompilerParams(use_tc_tiling_on_sc=False),
    )
    def body(text_hbm, seqv_hbm, table_hbm, out_hbm, idx_v, seq_v, *bufs):
        rows = list(bufs[:_NB])
        gsem = list(bufs[_NB:2 * _NB])
        wsem = list(bufs[2 * _NB:])
        wid = lax.axis_index("s") * _NC + lax.axis_index("c")
        base_row = wid * _PW

        pltpu.sync_copy(seqv_hbm, seq_v)
        seq = seq_v[...]
        lanes = lax.iota(jnp.int32, _L)

        pltpu.sync_copy(
            text_hbm.at[pl.ds(pl.multiple_of(base_row // _G, 8), _IR)], idx_v
        )

        def adj(r, carry):
            rbase = r * _G
            for i in range(_G // _L):
                v = idx_v[r, pl.ds(i * _L, _L)]
                pos = lax.rem(rbase + i * _L + lanes, _T)
                idx_v[r, pl.ds(i * _L, _L)] = jnp.where(pos < seq, v + 1, 0)
            return carry

        lax.fori_loop(0, _IR, adj, 0)

        def fire(q, b):
            for j in range(_QG):
                pltpu.async_copy(
                    table_hbm.at[idx_v.at[q * _QG + j]],
                    rows[b].at[pl.ds(j * _G, _G)],
                    gsem[b],
                )

        def wait_gathers(b):
            for j in range(_QG):
                pltpu.make_async_copy(
                    table_hbm.at[idx_v.at[0]],
                    rows[b].at[pl.ds(j * _G, _G)],
                    gsem[b],
                ).wait()

        def writeback(q, b):
            row0 = pl.multiple_of(base_row + q * _C, _C)
            pltpu.async_copy(rows[b], out_hbm.at[pl.ds(row0, _C)], wsem[b])

        def wait_writeback(b):
            pltpu.make_async_copy(
                rows[b],
                out_hbm.at[pl.ds(pl.multiple_of(base_row, _C), _C)],
                wsem[b],
            ).wait()

        fire(0, 0)

        def macro(gg, carry):
            for j in range(_NB):
                q = gg * _NB + j
                nb = (j + 1) % _NB

                @pl.when(q + 1 < _Q)
                def _():
                    @pl.when(q + 1 >= _NB)
                    def _():
                        wait_writeback(nb)

                    fire(q + 1, nb)

                wait_gathers(j)
                writeback(q, j)
            return carry

        lax.fori_loop(0, _Q // _NB, macro, 0)
        for b in range(_NB):
            wait_writeback(b)

    return body


_sc_gather = _make_sc_gather()


def kernel(text, seq_len, text_embed):
    text2d = text.reshape(_N // _G, _G)
    seqv = jnp.full((_L,), seq_len, dtype=jnp.int32)
    out = _sc_gather(text2d, seqv, text_embed)
    return out.reshape(_B, _T, _D)
